# Initial kernel scaffold; baseline (speedup 1.0000x reference)
import jax
import jax.numpy as jnp
from jax.experimental import pallas as pl


def kernel(long_history_data, params, epoch):
    raise NotImplementedError("write your pallas kernel here")


# speedup vs baseline: 1.7162x; 1.7162x over previous
import functools

import jax
import jax.numpy as jnp
from jax import lax
from jax.experimental import pallas as pl


def _ln(x, g, b):
    m = jnp.mean(x, axis=-1, keepdims=True)
    d = x - m
    v = jnp.mean(d * d, axis=-1, keepdims=True)
    return d * lax.rsqrt(v + 1e-5) * g + b


def _mm(a, b):
    return lax.dot_general(
        a, b, (((a.ndim - 1,), (0,)), ((), ())),
        preferred_element_type=jnp.float32)


def _imp_kernel(nv1_ref, nv2_ref, out_ref):
    a = jnp.maximum(
        jnp.dot(nv1_ref[:], nv2_ref[:], preferred_element_type=jnp.float32),
        0.0)
    m = jnp.max(a, axis=1, keepdims=True)
    e = jnp.exp(a - m)
    p = e / jnp.sum(e, axis=1, keepdims=True)
    out_ref[:] = jnp.sum(p, axis=0, keepdims=True)


def _embed_kernel(xr_ref, w_ref, b_ref, rw_ref, patches_ref, score_ref):
    t = pl.program_id(0)
    p = _mm(xr_ref[:], w_ref[:]) + b_ref[:]
    patches_ref[:] = p
    s1 = jnp.sum(jnp.abs(p), axis=2)
    contrib = jnp.sum(rw_ref[0] * s1, axis=0, keepdims=True)

    @pl.when(t == 0)
    def _():
        score_ref[:] = jnp.zeros_like(score_ref)

    score_ref[:] += contrib


def _mask_kernel(score_ref, out_ref, *, na, w, P):
    s = score_ref[:]
    anc = jnp.zeros(s.shape, jnp.bool_)
    for _ in range(na):
        m = jnp.max(s)
        sel = s == m
        anc = jnp.logical_or(anc, sel)
        s = jnp.where(sel, -jnp.inf, s)
    ti = lax.broadcasted_iota(jnp.int32, (P, P), 0)
    si = lax.broadcasted_iota(jnp.int32, (P, P), 1)
    allowed = (si <= ti) & (((ti - si) < w) | anc)
    out_ref[:] = jnp.where(allowed, 0.0, -1e9).astype(jnp.float32)


def _enc_kernel(x_ref, mask_ref, lng1_ref, lnb1_ref, wq_ref, bq_ref,
                wk_ref, bk_ref, wv_ref, bv_ref, wo_ref, bo_ref,
                lng2_ref, lnb2_ref, w1_ref, b1_ref, w2_ref, b2_ref,
                out_ref, *, depth, H):
    x = x_ref[:]
    mask = mask_ref[:]
    for d in range(depth):
        h1 = _ln(x, lng1_ref[d], lnb1_ref[d])
        acc = jnp.zeros_like(x)
        for hh in range(H):
            q = _mm(h1, wq_ref[d, hh]) + bq_ref[d, hh]
            k = _mm(h1, wk_ref[d, hh]) + bk_ref[d, hh]
            v = _mm(h1, wv_ref[d, hh]) + bv_ref[d, hh]
            s = lax.dot_general(
                q, k, (((2,), (2,)), ((0,), (0,))),
                preferred_element_type=jnp.float32) + mask
            m = jnp.max(s, axis=2, keepdims=True)
            e = jnp.exp(s - m)
            a = e / jnp.sum(e, axis=2, keepdims=True)
            o = lax.dot_general(
                a, v, (((2,), (1,)), ((0,), (0,))),
                preferred_element_type=jnp.float32)
            acc = acc + _mm(o, wo_ref[d, hh])
        x = x + acc + bo_ref[d]
        h2 = _ln(x, lng2_ref[d], lnb2_ref[d])
        mid = jax.nn.gelu(_mm(h2, w1_ref[d]) + b1_ref[d])
        x = x + _mm(mid, w2_ref[d]) + b2_ref[d]
    out_ref[0] = jnp.mean(x, axis=1)


def _fuse_kernel(p0_ref, p1_ref, fw_ref, fW_ref, fb_ref, fg_ref, fbt_ref,
                 out_ref):
    w = fw_ref[:]
    e = jnp.exp(w - jnp.max(w))
    sm = e / jnp.sum(e)
    f = p0_ref[:] * sm[0, 0] + p1_ref[:] * sm[0, 1]
    f = _mm(f, fW_ref[:]) + fb_ref[:]
    out_ref[:] = _ln(f, fg_ref[:], fbt_ref[:])


def _pick_tile(n, cap=32):
    for g in range(cap, 0, -1):
        if n % g == 0:
            return g
    return 1


def kernel(long_history_data, params, epoch):
    del epoch
    x = long_history_data
    Bn, L, N, C = x.shape
    D = params['fuse_W'].shape[0]
    H = 4
    dh = D // H
    patch_sizes = [int(W.shape[0]) // C for W in params['pe_W']]
    BN = Bn * N
    G = _pick_tile(BN)
    T = BN // G

    f32 = jnp.float32

    imp = pl.pallas_call(
        _imp_kernel,
        out_shape=jax.ShapeDtypeStruct((1, N), f32),
    )(params['nodevec1'], params['nodevec2'])

    rw = jnp.tile(imp, (Bn, 1)).reshape(T, G, 1)

    xt = jnp.transpose(x, (0, 2, 3, 1))

    pooled = []
    for i, ps in enumerate(patch_sizes):
        P = L // ps
        K = ps * C
        xr = (xt.reshape(Bn, N, C, P, ps)
                .transpose(0, 1, 3, 4, 2)
                .reshape(BN, P, K))
        b2d = params['pe_b'][i].reshape(1, D)

        patches, score = pl.pallas_call(
            _embed_kernel,
            grid=(T,),
            in_specs=[
                pl.BlockSpec((G, P, K), lambda t: (t, 0, 0)),
                pl.BlockSpec((K, D), lambda t: (0, 0)),
                pl.BlockSpec((1, D), lambda t: (0, 0)),
                pl.BlockSpec((1, G, 1), lambda t: (t, 0, 0)),
            ],
            out_specs=[
                pl.BlockSpec((G, P, D), lambda t: (t, 0, 0)),
                pl.BlockSpec((1, P), lambda t: (0, 0)),
            ],
            out_shape=[
                jax.ShapeDtypeStruct((BN, P, D), f32),
                jax.ShapeDtypeStruct((1, P), f32),
            ],
        )(xr, params['pe_W'][i], b2d, rw)

        na = max(1, int(0.1 * P))
        w = max(1, ps // 4)
        maskadd = pl.pallas_call(
            functools.partial(_mask_kernel, na=na, w=w, P=P),
            out_shape=jax.ShapeDtypeStruct((P, P), f32),
        )(score)

        blocks = params['encoders'][i]
        depth = len(blocks)
        scale = 1.0 / (dh ** 0.5)
        lng1 = jnp.stack([b['ln1g'].reshape(1, D) for b in blocks])
        lnb1 = jnp.stack([b['ln1b'].reshape(1, D) for b in blocks])
        wq = jnp.stack([(b['Wq'] * scale).reshape(D, H, dh).transpose(1, 0, 2)
                        for b in blocks])
        bq = jnp.stack([(b['bq'] * scale).reshape(H, 1, dh) for b in blocks])
        wk = jnp.stack([b['Wk'].reshape(D, H, dh).transpose(1, 0, 2)
                        for b in blocks])
        bk = jnp.stack([b['bk'].reshape(H, 1, dh) for b in blocks])
        wv = jnp.stack([b['Wv'].reshape(D, H, dh).transpose(1, 0, 2)
                        for b in blocks])
        bv = jnp.stack([b['bv'].reshape(H, 1, dh) for b in blocks])
        wo = jnp.stack([b['Wo'].reshape(H, dh, D) for b in blocks])
        bo = jnp.stack([b['bo'].reshape(1, D) for b in blocks])
        lng2 = jnp.stack([b['ln2g'].reshape(1, D) for b in blocks])
        lnb2 = jnp.stack([b['ln2b'].reshape(1, D) for b in blocks])
        w1 = jnp.stack([b['W1'] for b in blocks])
        b1 = jnp.stack([b['b1'].reshape(1, -1) for b in blocks])
        w2 = jnp.stack([b['W2'] for b in blocks])
        b2 = jnp.stack([b['b2'].reshape(1, D) for b in blocks])
        MD = w1.shape[-1]

        cst = lambda *dims: pl.BlockSpec(dims, lambda t: (0,) * len(dims))
        pooled_i = pl.pallas_call(
            functools.partial(_enc_kernel, depth=depth, H=H),
            grid=(T,),
            in_specs=[
                pl.BlockSpec((G, P, D), lambda t: (t, 0, 0)),
                cst(P, P),
                cst(depth, 1, D), cst(depth, 1, D),
                cst(depth, H, D, dh), cst(depth, H, 1, dh),
                cst(depth, H, D, dh), cst(depth, H, 1, dh),
                cst(depth, H, D, dh), cst(depth, H, 1, dh),
                cst(depth, H, dh, D), cst(depth, 1, D),
                cst(depth, 1, D), cst(depth, 1, D),
                cst(depth, D, MD), cst(depth, 1, MD),
                cst(depth, MD, D), cst(depth, 1, D),
            ],
            out_specs=pl.BlockSpec((1, G, D), lambda t: (t, 0, 0)),
            out_shape=jax.ShapeDtypeStruct((T, G, D), f32),
        )(patches, maskadd, lng1, lnb1, wq, bq, wk, bk, wv, bv, wo, bo,
          lng2, lnb2, w1, b1, w2, b2)
        pooled.append(pooled_i)

    final = pl.pallas_call(
        _fuse_kernel,
        out_shape=jax.ShapeDtypeStruct((T, G, D), f32),
    )(pooled[0], pooled[1], params['fusion_w'].reshape(1, -1),
      params['fuse_W'], params['fuse_b'].reshape(1, D),
      params['fin_g'].reshape(1, D), params['fin_b'].reshape(1, D))

    return final.reshape(Bn, N, D)


# speedup vs baseline: 1.9582x; 1.1410x over previous
import functools

import jax
import jax.numpy as jnp
from jax import lax
from jax.experimental import pallas as pl


def _ln(x, g, b):
    m = jnp.mean(x, axis=-1, keepdims=True)
    d = x - m
    v = jnp.mean(d * d, axis=-1, keepdims=True)
    return d * lax.rsqrt(v + 1e-5) * g + b


def _mm(a, b):
    return lax.dot_general(
        a, b, (((a.ndim - 1,), (0,)), ((), ())),
        preferred_element_type=jnp.float32)


def _mmb(a, b):
    return lax.dot_general(
        a.astype(jnp.bfloat16), b,
        (((a.ndim - 1,), (0,)), ((), ())),
        preferred_element_type=jnp.float32)


def _imp_kernel(nv1_ref, nv2_ref, out_ref):
    a = jnp.maximum(
        jnp.dot(nv1_ref[:].astype(jnp.bfloat16),
                nv2_ref[:].astype(jnp.bfloat16),
                preferred_element_type=jnp.float32),
        0.0)
    m = jnp.max(a, axis=1, keepdims=True)
    e = jnp.exp(a - m)
    p = e / jnp.sum(e, axis=1, keepdims=True)
    out_ref[:] = jnp.sum(p, axis=0, keepdims=True)


def _embed_kernel(xr_ref, w_ref, b_ref, patches_ref, r1_ref):
    p = _mm(xr_ref[:], w_ref[:]) + b_ref[:]
    patches_ref[:] = p
    r1_ref[0] = jnp.sum(jnp.abs(p), axis=2)


def _mask_kernel(r3_ref, imp_ref, out_ref, *, na, w, P):
    M = r3_ref[0] + r3_ref[1]
    s = jnp.dot(imp_ref[:].astype(jnp.bfloat16),
                M.astype(jnp.bfloat16),
                preferred_element_type=jnp.float32)
    anc = jnp.zeros(s.shape, jnp.bool_)
    for _ in range(na):
        m = jnp.max(s)
        sel = s == m
        anc = jnp.logical_or(anc, sel)
        s = jnp.where(sel, -jnp.inf, s)
    ti = lax.broadcasted_iota(jnp.int32, (P, P), 0)
    si = lax.broadcasted_iota(jnp.int32, (P, P), 1)
    allowed = (si <= ti) & (((ti - si) < w) | anc)
    out_ref[:] = jnp.where(allowed, 0.0, -1e9).astype(jnp.float32)


def _enc_kernel(x_ref, mask_ref, lng1_ref, lnb1_ref, wq_ref, bq_ref,
                wk_ref, bk_ref, wv_ref, bv_ref, wo_ref, bo_ref,
                lng2_ref, lnb2_ref, w1_ref, b1_ref, w2_ref, b2_ref,
                out_ref, *, depth, H):
    x = x_ref[:]
    mask = mask_ref[:]
    for d in range(depth):
        h1 = _ln(x, lng1_ref[d], lnb1_ref[d])
        acc = jnp.zeros_like(x)
        for hh in range(H):
            q = _mmb(h1, wq_ref[d, hh]) + bq_ref[d, hh]
            k = _mmb(h1, wk_ref[d, hh]) + bk_ref[d, hh]
            v = _mmb(h1, wv_ref[d, hh]) + bv_ref[d, hh]
            s = lax.dot_general(
                q.astype(jnp.bfloat16), k.astype(jnp.bfloat16),
                (((2,), (2,)), ((0,), (0,))),
                preferred_element_type=jnp.float32) + mask
            m = jnp.max(s, axis=2, keepdims=True)
            e = jnp.exp(s - m)
            a = e / jnp.sum(e, axis=2, keepdims=True)
            o = lax.dot_general(
                a.astype(jnp.bfloat16), v.astype(jnp.bfloat16),
                (((2,), (1,)), ((0,), (0,))),
                preferred_element_type=jnp.float32)
            acc = acc + _mmb(o, wo_ref[d, hh])
        x = x + acc + bo_ref[d]
        h2 = _ln(x, lng2_ref[d], lnb2_ref[d])
        mid = jax.nn.gelu(_mmb(h2, w1_ref[d]) + b1_ref[d])
        x = x + _mmb(mid, w2_ref[d]) + b2_ref[d]
    out_ref[0] = jnp.mean(x, axis=1)


def _fuse_kernel(p0_ref, p1_ref, fw_ref, fW_ref, fb_ref, fg_ref, fbt_ref,
                 out_ref):
    w = fw_ref[:]
    e = jnp.exp(w - jnp.max(w))
    sm = e / jnp.sum(e)
    f = p0_ref[:] * sm[0, 0] + p1_ref[:] * sm[0, 1]
    f = _mmb(f, fW_ref[:]) + fb_ref[:]
    out_ref[:] = _ln(f, fg_ref[:], fbt_ref[:])


def _pick_tile(n, cap=32):
    for g in range(cap, 0, -1):
        if n % g == 0:
            return g
    return 1


def kernel(long_history_data, params, epoch):
    del epoch
    x = long_history_data
    Bn, L, N, C = x.shape
    D = params['fuse_W'].shape[0]
    H = 4
    dh = D // H
    patch_sizes = [int(W.shape[0]) // C for W in params['pe_W']]
    BN = Bn * N
    G = _pick_tile(BN)
    T = BN // G

    f32 = jnp.float32

    imp = pl.pallas_call(
        _imp_kernel,
        out_shape=jax.ShapeDtypeStruct((1, N), f32),
    )(params['nodevec1'], params['nodevec2'])

    xt = jnp.transpose(x, (0, 2, 3, 1))

    pooled = []
    for i, ps in enumerate(patch_sizes):
        P = L // ps
        K = ps * C
        xr = (xt.reshape(Bn, N, C, P, ps)
                .transpose(0, 1, 3, 4, 2)
                .reshape(BN, P, K)).astype(jnp.bfloat16)
        b2d = params['pe_b'][i].reshape(1, D)

        patches, r1 = pl.pallas_call(
            _embed_kernel,
            grid=(T,),
            in_specs=[
                pl.BlockSpec((G, P, K), lambda t: (t, 0, 0)),
                pl.BlockSpec((K, D), lambda t: (0, 0)),
                pl.BlockSpec((1, D), lambda t: (0, 0)),
            ],
            out_specs=[
                pl.BlockSpec((G, P, D), lambda t: (t, 0, 0)),
                pl.BlockSpec((1, G, P), lambda t: (t, 0, 0)),
            ],
            out_shape=[
                jax.ShapeDtypeStruct((BN, P, D), f32),
                jax.ShapeDtypeStruct((T, G, P), f32),
            ],
        )(xr, params['pe_W'][i].astype(jnp.bfloat16), b2d)

        na = max(1, int(0.1 * P))
        w = max(1, ps // 4)
        maskadd = pl.pallas_call(
            functools.partial(_mask_kernel, na=na, w=w, P=P),
            out_shape=jax.ShapeDtypeStruct((P, P), f32),
        )(r1.reshape(Bn, N, P), imp)

        blocks = params['encoders'][i]
        depth = len(blocks)
        scale = 1.0 / (dh ** 0.5)
        lng1 = jnp.stack([b['ln1g'].reshape(1, D) for b in blocks])
        lnb1 = jnp.stack([b['ln1b'].reshape(1, D) for b in blocks])
        bf16 = jnp.bfloat16
        wq = jnp.stack([(b['Wq'] * scale).reshape(D, H, dh).transpose(1, 0, 2)
                        for b in blocks]).astype(bf16)
        bq = jnp.stack([(b['bq'] * scale).reshape(H, 1, dh) for b in blocks])
        wk = jnp.stack([b['Wk'].reshape(D, H, dh).transpose(1, 0, 2)
                        for b in blocks]).astype(bf16)
        bk = jnp.stack([b['bk'].reshape(H, 1, dh) for b in blocks])
        wv = jnp.stack([b['Wv'].reshape(D, H, dh).transpose(1, 0, 2)
                        for b in blocks]).astype(bf16)
        bv = jnp.stack([b['bv'].reshape(H, 1, dh) for b in blocks])
        wo = jnp.stack([b['Wo'].reshape(H, dh, D) for b in blocks]).astype(bf16)
        bo = jnp.stack([b['bo'].reshape(1, D) for b in blocks])
        lng2 = jnp.stack([b['ln2g'].reshape(1, D) for b in blocks])
        lnb2 = jnp.stack([b['ln2b'].reshape(1, D) for b in blocks])
        w1 = jnp.stack([b['W1'] for b in blocks]).astype(bf16)
        b1 = jnp.stack([b['b1'].reshape(1, -1) for b in blocks])
        w2 = jnp.stack([b['W2'] for b in blocks]).astype(bf16)
        b2 = jnp.stack([b['b2'].reshape(1, D) for b in blocks])
        MD = w1.shape[-1]

        cst = lambda *dims: pl.BlockSpec(dims, lambda t: (0,) * len(dims))
        pooled_i = pl.pallas_call(
            functools.partial(_enc_kernel, depth=depth, H=H),
            grid=(T,),
            in_specs=[
                pl.BlockSpec((G, P, D), lambda t: (t, 0, 0)),
                cst(P, P),
                cst(depth, 1, D), cst(depth, 1, D),
                cst(depth, H, D, dh), cst(depth, H, 1, dh),
                cst(depth, H, D, dh), cst(depth, H, 1, dh),
                cst(depth, H, D, dh), cst(depth, H, 1, dh),
                cst(depth, H, dh, D), cst(depth, 1, D),
                cst(depth, 1, D), cst(depth, 1, D),
                cst(depth, D, MD), cst(depth, 1, MD),
                cst(depth, MD, D), cst(depth, 1, D),
            ],
            out_specs=pl.BlockSpec((1, G, D), lambda t: (t, 0, 0)),
            out_shape=jax.ShapeDtypeStruct((T, G, D), f32),
        )(patches, maskadd, lng1, lnb1, wq, bq, wk, bk, wv, bv, wo, bo,
          lng2, lnb2, w1, b1, w2, b2)
        pooled.append(pooled_i)

    final = pl.pallas_call(
        _fuse_kernel,
        out_shape=jax.ShapeDtypeStruct((T, G, D), f32),
    )(pooled[0], pooled[1], params['fusion_w'].reshape(1, -1),
      params['fuse_W'].astype(jnp.bfloat16), params['fuse_b'].reshape(1, D),
      params['fin_g'].reshape(1, D), params['fin_b'].reshape(1, D))

    return final.reshape(Bn, N, D)
